# Initial kernel scaffold; baseline (speedup 1.0000x reference)
import jax
import jax.numpy as jnp
from jax.experimental import pallas as pl


def kernel(x, w1, b1, w2, b2, wmu, bmu, wsd, bsd):
    raise NotImplementedError("write your pallas kernel here")


# speedup vs baseline: 1.3303x; 1.3303x over previous
import jax
import jax.numpy as jnp
from jax.experimental import pallas as pl
from jax.experimental.pallas import tpu as pltpu


def _policy_mlp_kernel(x_ref, w1_ref, b1_ref, w2_ref, b2_ref, wh_ref, bh_ref,
                       mu_ref, sd_ref):
    x = x_ref[...]
    w1 = w1_ref[...]

    h1 = (x[:, 0:1] * w1[0:1, :]
          + x[:, 1:2] * w1[1:2, :]
          + x[:, 2:3] * w1[2:3, :]
          + b1_ref[...])
    h1 = jnp.maximum(h1, 0.0).astype(jnp.bfloat16)

    h2 = jnp.dot(h1, w2_ref[...], preferred_element_type=jnp.float32)
    h2 = jnp.maximum(h2 + b2_ref[...], 0.0).astype(jnp.bfloat16)

    y = jnp.dot(h2, wh_ref[...], preferred_element_type=jnp.float32) + bh_ref[...]

    mu_ref[...] = 2.0 * jnp.tanh(y[:, 0:1])
    sd_ref[...] = jnp.logaddexp(y[:, 1:2], 0.0) + 0.001


def _round_up(n, m):
    return ((n + m - 1) // m) * m


def kernel(x, w1, b1, w2, b2, wmu, bmu, wsd, bsd, *, tile_b=2048):
    B, F = x.shape
    assert F == 3

    if B <= 256:
        tb = max(8, _round_up(B, 8))
    else:
        half = _round_up(_round_up(B, 256) // 2, 256)
        tb = max(256, min(_round_up(tile_b, 256), half))
    Bp = pl.cdiv(B, tb) * tb
    xp = jnp.pad(x, ((0, Bp - B), (0, 0))) if Bp != B else x

    w1 = w1.astype(jnp.float32)
    b1 = b1.astype(jnp.float32)
    w2b = w2.astype(jnp.bfloat16)
    b2 = b2.astype(jnp.float32)
    wh = jnp.concatenate([wmu, wsd], axis=1).astype(jnp.bfloat16)
    bh = jnp.concatenate([bmu, bsd], axis=1).astype(jnp.float32)

    batch_map = lambda i: (i, 0)
    const_map = lambda i: (0, 0)

    weight_bytes = 4 * (3 * 64 + 64 + 256 + 2) + 2 * (64 * 256 + 256 * 2)
    cost = pl.CostEstimate(
        flops=2 * Bp * (3 * 64 + 64 * 256 + 256 * 2),
        transcendentals=2 * Bp,
        bytes_accessed=20 * Bp + weight_bytes,
    )

    mu, sd = pl.pallas_call(
        _policy_mlp_kernel,
        out_shape=(
            jax.ShapeDtypeStruct((Bp, 1), jnp.float32),
            jax.ShapeDtypeStruct((Bp, 1), jnp.float32),
        ),
        grid=(Bp // tb,),
        in_specs=[
            pl.BlockSpec((tb, 3), batch_map),
            pl.BlockSpec((3, 64), const_map),
            pl.BlockSpec((1, 64), const_map),
            pl.BlockSpec((64, 256), const_map),
            pl.BlockSpec((1, 256), const_map),
            pl.BlockSpec((256, 2), const_map),
            pl.BlockSpec((1, 2), const_map),
        ],
        out_specs=(
            pl.BlockSpec((tb, 1), batch_map),
            pl.BlockSpec((tb, 1), batch_map),
        ),
        compiler_params=pltpu.CompilerParams(
            dimension_semantics=("parallel",),
        ),
        cost_estimate=cost,
    )(xp, w1, b1, w2b, b2, wh, bh)

    return mu[:B], sd[:B]


# speedup vs baseline: 8.1444x; 6.1221x over previous
import jax
import jax.numpy as jnp
from jax.experimental import pallas as pl
from jax.experimental.pallas import tpu as pltpu

_BF = jnp.bfloat16
_F32 = jnp.float32


def _policy_kernel(xt_ref, w1s_ref, w2s_ref, wht_ref, bht_ref, mu_ref, sd_ref):
    xt = xt_ref[...]
    tbl = xt.shape[1]
    x_hi = xt.astype(_BF)
    x_lo = (xt - x_hi.astype(_F32)).astype(_BF)
    ones2 = jnp.ones((2, tbl), _BF)

    xs = jnp.concatenate([x_hi, x_hi, x_lo, ones2], axis=0)
    h1 = jnp.dot(w1s_ref[...], xs, preferred_element_type=_F32)
    h1 = jnp.maximum(h1, 0.0)

    h_hi = h1.astype(_BF)
    h_lo = (h1 - h_hi.astype(_F32)).astype(_BF)
    hs = jnp.concatenate([h_hi, h_hi, h_lo, ones2], axis=0)
    h2 = jnp.dot(w2s_ref[...], hs, preferred_element_type=_F32)
    h2 = jnp.maximum(h2, 0.0)

    y = jnp.dot(wht_ref[...], h2, preferred_element_type=_F32) + bht_ref[...]

    mu_ref[...] = 2.0 * jnp.tanh(y[0:1, :])
    sd_ref[...] = jnp.logaddexp(y[1:2, :], 0.0) + 0.001


def _hi_lo(a):
    hi = a.astype(_BF)
    lo = (a - hi.astype(_F32)).astype(_BF)
    return hi, lo


def _round_up(n, m):
    return ((n + m - 1) // m) * m


def kernel(x, w1, b1, w2, b2, wmu, bmu, wsd, bsd, *, tile_b=8192):
    B, F = x.shape
    assert F == 3

    tbl = max(128, min(_round_up(tile_b, 128), _round_up(B, 128)))
    Bp = pl.cdiv(B, tbl) * tbl
    xp = jnp.pad(x, ((0, Bp - B), (0, 0))) if Bp != B else x
    xt = xp.T

    w1_hi, w1_lo = _hi_lo(w1.T.astype(_F32))
    b1_hi, b1_lo = _hi_lo(b1.reshape(-1, 1).astype(_F32))
    w1s = jnp.concatenate([w1_hi, w1_lo, w1_hi, b1_hi, b1_lo], axis=1)

    w2_hi, w2_lo = _hi_lo(w2.T.astype(_F32))
    b2_hi, b2_lo = _hi_lo(b2.reshape(-1, 1).astype(_F32))
    w2s = jnp.concatenate([w2_hi, w2_lo, w2_hi, b2_hi, b2_lo], axis=1)

    wht = jnp.concatenate([wmu, wsd], axis=1).T.astype(_F32)
    bht = jnp.concatenate([bmu, bsd], axis=1).T.astype(_F32)

    batch_map = lambda i: (0, i)
    const_map = lambda i: (0, 0)

    weight_bytes = 2 * (64 * 11 + 256 * 194) + 4 * (2 * 256 + 2)
    cost = pl.CostEstimate(
        flops=2 * Bp * (3 * 64 + 64 * 256 + 256 * 2),
        transcendentals=2 * Bp,
        bytes_accessed=20 * Bp + weight_bytes,
    )

    mu, sd = pl.pallas_call(
        _policy_kernel,
        out_shape=(
            jax.ShapeDtypeStruct((1, Bp), _F32),
            jax.ShapeDtypeStruct((1, Bp), _F32),
        ),
        grid=(Bp // tbl,),
        in_specs=[
            pl.BlockSpec((3, tbl), batch_map),
            pl.BlockSpec((64, 11), const_map),
            pl.BlockSpec((256, 194), const_map),
            pl.BlockSpec((2, 256), const_map),
            pl.BlockSpec((2, 1), const_map),
        ],
        out_specs=(
            pl.BlockSpec((1, tbl), batch_map),
            pl.BlockSpec((1, tbl), batch_map),
        ),
        compiler_params=pltpu.CompilerParams(
            dimension_semantics=("parallel",),
        ),
        cost_estimate=cost,
    )(xt, w1s, w2s, wht, bht)

    return mu.reshape(Bp, 1)[:B], sd.reshape(Bp, 1)[:B]


# speedup vs baseline: 8.8202x; 1.0830x over previous
import jax
import jax.numpy as jnp
from jax.experimental import pallas as pl
from jax.experimental.pallas import tpu as pltpu

_BF = jnp.bfloat16
_F32 = jnp.float32


def _policy_kernel(xt_ref, w1s_ref, w2s_ref, wht_ref, bht_ref, mu_ref, sd_ref):
    xt = xt_ref[...]
    tbl = xt.shape[1]
    x_hi = xt.astype(_BF)
    x_lo = (xt - x_hi.astype(_F32)).astype(_BF)
    ones2 = jnp.ones((2, tbl), _BF)

    xs = jnp.concatenate([x_hi, x_hi, x_lo, ones2], axis=0)
    h1 = jnp.dot(w1s_ref[...], xs, preferred_element_type=_F32)
    h1 = jnp.maximum(h1, 0.0)

    h_hi = h1.astype(_BF)
    hs = jnp.concatenate([h_hi, h_hi, ones2], axis=0)
    h2 = jnp.dot(w2s_ref[...], hs, preferred_element_type=_F32)
    h2 = jnp.maximum(h2, 0.0)

    y = jnp.dot(wht_ref[...], h2, preferred_element_type=_F32) + bht_ref[...]

    mu_ref[...] = 2.0 * jnp.tanh(y[0:1, :])
    sd_ref[...] = jnp.logaddexp(y[1:2, :], 0.0) + 0.001


def _hi_lo(a):
    hi = a.astype(_BF)
    lo = (a - hi.astype(_F32)).astype(_BF)
    return hi, lo


def _round_up(n, m):
    return ((n + m - 1) // m) * m


def kernel(x, w1, b1, w2, b2, wmu, bmu, wsd, bsd, *, tile_b=16384):
    B, F = x.shape
    assert F == 3

    tbl = max(128, min(_round_up(tile_b, 128), _round_up(B, 128)))
    Bp = pl.cdiv(B, tbl) * tbl
    xp = jnp.pad(x, ((0, Bp - B), (0, 0))) if Bp != B else x
    xt = xp.T

    w1_hi, w1_lo = _hi_lo(w1.T.astype(_F32))
    b1_hi, b1_lo = _hi_lo(b1.reshape(-1, 1).astype(_F32))
    w1s = jnp.concatenate([w1_hi, w1_lo, w1_hi, b1_hi, b1_lo], axis=1)

    w2_hi, w2_lo = _hi_lo(w2.T.astype(_F32))
    b2_hi, b2_lo = _hi_lo(b2.reshape(-1, 1).astype(_F32))
    w2s = jnp.concatenate([w2_hi, w2_lo, b2_hi, b2_lo], axis=1)

    wht = jnp.concatenate([wmu, wsd], axis=1).T.astype(_F32)
    bht = jnp.concatenate([bmu, bsd], axis=1).T.astype(_F32)

    batch_map = lambda i: (0, i)
    const_map = lambda i: (0, 0)

    weight_bytes = 2 * (64 * 11 + 256 * 194) + 4 * (2 * 256 + 2)
    cost = pl.CostEstimate(
        flops=2 * Bp * (3 * 64 + 64 * 256 + 256 * 2),
        transcendentals=2 * Bp,
        bytes_accessed=20 * Bp + weight_bytes,
    )

    mu, sd = pl.pallas_call(
        _policy_kernel,
        out_shape=(
            jax.ShapeDtypeStruct((1, Bp), _F32),
            jax.ShapeDtypeStruct((1, Bp), _F32),
        ),
        grid=(Bp // tbl,),
        in_specs=[
            pl.BlockSpec((3, tbl), batch_map),
            pl.BlockSpec((64, 11), const_map),
            pl.BlockSpec((256, 130), const_map),
            pl.BlockSpec((2, 256), const_map),
            pl.BlockSpec((2, 1), const_map),
        ],
        out_specs=(
            pl.BlockSpec((1, tbl), batch_map),
            pl.BlockSpec((1, tbl), batch_map),
        ),
        compiler_params=pltpu.CompilerParams(
            dimension_semantics=("parallel",),
        ),
        cost_estimate=cost,
    )(xt, w1s, w2s, wht, bht)

    return mu.reshape(Bp, 1)[:B], sd.reshape(Bp, 1)[:B]
